# baseline (speedup 1.0000x reference)
import functools

import jax
import jax.numpy as jnp
from jax import lax
from jax.experimental import pallas as pl
from jax.experimental.pallas import tpu as pltpu
from jax.experimental.pallas import tpu_sc as plsc

_TOP_K = 8
_BT = 2048
_CHUNKS = 2
_NC, _NS, _L = 2, 16, 16
_NW = _NC * _NS
_GI = 2

_SORT8 = [(0, 1), (2, 3), (4, 5), (6, 7), (0, 2), (1, 3), (4, 6), (5, 7),
          (1, 2), (5, 6), (0, 4), (1, 5), (2, 6), (3, 7), (2, 4), (3, 5),
          (1, 2), (3, 4), (5, 6)]
_MERGE8 = [(0, 4), (1, 5), (2, 6), (3, 7), (0, 2), (1, 3), (4, 6), (5, 7),
           (0, 1), (2, 3), (4, 5), (6, 7)]


def _keys_block(x_ref, wt_ref, k_out_ref):
    logits = lax.dot_general(
        x_ref[...], wt_ref[...], (((1,), (0,)), ((), ())),
        preferred_element_type=jnp.float32,
    )
    m = jnp.max(logits, axis=1, keepdims=True)
    e = jnp.exp(logits - m)
    s = jnp.sum(e, axis=1, keepdims=True)
    p = e / s
    ncol = logits.shape[1]
    col = lax.broadcasted_iota(jnp.int32, logits.shape, 1)
    keys = (lax.bitcast_convert_type(p, jnp.int32) & ~63) | (ncol - 1 - col)
    k_out_ref[...] = keys.T


def _tc_keys_t(x, wt, chunk, nchunks):
    tokens, hidden = x.shape
    nexp = wt.shape[1]
    per_chunk = tokens // nchunks
    blocks = per_chunk // _BT
    base = chunk * blocks
    return pl.pallas_call(
        _keys_block,
        grid=(blocks,),
        in_specs=[
            pl.BlockSpec((_BT, hidden), lambda i: (base + i, 0)),
            pl.BlockSpec((hidden, nexp), lambda i: (0, 0)),
        ],
        out_specs=pl.BlockSpec((nexp, _BT), lambda i: (0, i)),
        out_shape=jax.ShapeDtypeStruct((nexp, per_chunk), jnp.int32),
        compiler_params=pltpu.CompilerParams(
            dimension_semantics=("arbitrary",),
        ),
    )(x, wt)


def _sc_topk_t(keys_t):
    nexp, rows = keys_t.shape
    rpw = rows // _NW
    groups = rpw // _L
    mesh = plsc.VectorSubcoreMesh(core_axis_name="c", subcore_axis_name="s")

    @functools.partial(
        pl.kernel,
        mesh=mesh,
        out_type=[
            jax.ShapeDtypeStruct((_TOP_K, rows), jnp.float32),
            jax.ShapeDtypeStruct((_TOP_K, rows), jnp.int32),
        ],
        scratch_types=[
            pltpu.VMEM((nexp * rpw,), jnp.int32),
            pltpu.VMEM((_TOP_K * rpw,), jnp.float32),
            pltpu.VMEM((_TOP_K * rpw,), jnp.int32),
            pltpu.SemaphoreType.DMA,
        ],
    )
    def k(keys_hbm, w_hbm, i_hbm, kbuf, wbuf, ibuf, sem):
        wid = lax.axis_index("s") * _NC + lax.axis_index("c")
        base = wid * rpw
        copies = [
            pltpu.async_copy(
                keys_hbm.at[e, pl.ds(base, rpw)],
                kbuf.at[pl.ds(e * rpw, rpw)], sem)
            for e in range(nexp)
        ]
        for c in copies:
            c.wait()
        def load_sorted8(b, g0, gi):
            s = [kbuf[pl.ds((b * _TOP_K + j) * rpw + (g0 + gi) * _L, _L)]
                 for j in range(_TOP_K)]
            for (i, j) in _SORT8:
                hi = jnp.maximum(s[i], s[j])
                lo = jnp.minimum(s[i], s[j])
                s[i], s[j] = hi, lo
            return s

        def group_body(gb, carry):
            g0 = gb * _GI

            def ebatch(b, ts):
                new = []
                for gi, t in enumerate(ts):
                    s = load_sorted8(b, g0, gi)
                    m = [jnp.maximum(t[i], s[7 - i]) for i in range(_TOP_K)]
                    for (i, j) in _MERGE8:
                        hi = jnp.maximum(m[i], m[j])
                        lo = jnp.minimum(m[i], m[j])
                        m[i], m[j] = hi, lo
                    new.append(tuple(m))
                return tuple(new)

            init = tuple(tuple(load_sorted8(0, g0, gi)) for gi in range(_GI))
            ts = lax.fori_loop(1, nexp // _TOP_K, ebatch, init)
            for gi, t in enumerate(ts):
                opos = (g0 + gi) * _L
                for j in range(_TOP_K):
                    w = lax.bitcast_convert_type(t[j] & ~63, jnp.float32)
                    i = (nexp - 1) - (t[j] & 63)
                    wbuf[pl.ds(j * rpw + opos, _L)] = w
                    ibuf[pl.ds(j * rpw + opos, _L)] = i
            return carry

        lax.fori_loop(0, groups // _GI, group_body, 0)
        for j in range(_TOP_K):
            pltpu.sync_copy(wbuf.at[pl.ds(j * rpw, rpw)],
                            w_hbm.at[j, pl.ds(base, rpw)])
            pltpu.sync_copy(ibuf.at[pl.ds(j * rpw, rpw)],
                            i_hbm.at[j, pl.ds(base, rpw)])

    return k(keys_t)


def kernel(x, weight):
    wt = weight.T
    ws, idxs = [], []
    for c in range(_CHUNKS):
        keys_t = _tc_keys_t(x, wt, c, _CHUNKS)
        w_t, i_t = _sc_topk_t(keys_t)
        ws.append(w_t.T)
        idxs.append(i_t.T)
    return jnp.concatenate(ws, axis=0), jnp.concatenate(idxs, axis=0)


# speedup vs baseline: 1.0789x; 1.0789x over previous
import functools

import jax
import jax.numpy as jnp
from jax import lax
from jax.experimental import pallas as pl
from jax.experimental.pallas import tpu as pltpu
from jax.experimental.pallas import tpu_sc as plsc

_TOP_K = 8
_BT = 2048
_CHUNKS = 1
_NC, _NS, _L = 2, 16, 16
_NW = _NC * _NS
_GI = 2

_SORT8 = [(0, 1), (2, 3), (4, 5), (6, 7), (0, 2), (1, 3), (4, 6), (5, 7),
          (1, 2), (5, 6), (0, 4), (1, 5), (2, 6), (3, 7), (2, 4), (3, 5),
          (1, 2), (3, 4), (5, 6)]
_MERGE8 = [(0, 4), (1, 5), (2, 6), (3, 7), (0, 2), (1, 3), (4, 6), (5, 7),
           (0, 1), (2, 3), (4, 5), (6, 7)]


def _keys_block(x_ref, wt_ref, k_out_ref):
    logits = lax.dot_general(
        x_ref[...], wt_ref[...], (((1,), (0,)), ((), ())),
        preferred_element_type=jnp.float32,
    )
    m = jnp.max(logits, axis=1, keepdims=True)
    e = jnp.exp(logits - m)
    s = jnp.sum(e, axis=1, keepdims=True)
    p = e / s
    ncol = logits.shape[1]
    col = lax.broadcasted_iota(jnp.int32, logits.shape, 1)
    keys = (lax.bitcast_convert_type(p, jnp.int32) & ~63) | (ncol - 1 - col)
    k_out_ref[...] = keys.T


def _tc_keys_t(x, wt, chunk, nchunks):
    tokens, hidden = x.shape
    nexp = wt.shape[1]
    per_chunk = tokens // nchunks
    blocks = per_chunk // _BT
    base = chunk * blocks
    return pl.pallas_call(
        _keys_block,
        grid=(blocks,),
        in_specs=[
            pl.BlockSpec((_BT, hidden), lambda i: (base + i, 0)),
            pl.BlockSpec((hidden, nexp), lambda i: (0, 0)),
        ],
        out_specs=pl.BlockSpec((nexp, _BT), lambda i: (0, i)),
        out_shape=jax.ShapeDtypeStruct((nexp, per_chunk), jnp.int32),
        compiler_params=pltpu.CompilerParams(
            dimension_semantics=("arbitrary",),
        ),
    )(x, wt)


def _sc_topk_t(keys_t):
    nexp, rows = keys_t.shape
    rpw = rows // _NW
    groups = rpw // _L
    mesh = plsc.VectorSubcoreMesh(core_axis_name="c", subcore_axis_name="s")

    @functools.partial(
        pl.kernel,
        mesh=mesh,
        out_type=[
            jax.ShapeDtypeStruct((_TOP_K, rows), jnp.float32),
            jax.ShapeDtypeStruct((_TOP_K, rows), jnp.int32),
        ],
        scratch_types=[
            pltpu.VMEM((nexp * rpw,), jnp.int32),
            pltpu.VMEM((_TOP_K * rpw,), jnp.float32),
            pltpu.VMEM((_TOP_K * rpw,), jnp.int32),
            pltpu.SemaphoreType.DMA,
            pltpu.SemaphoreType.DMA,
        ],
    )
    def k(keys_hbm, w_hbm, i_hbm, kbuf, wbuf, ibuf, sem0, sem1):
        wid = lax.axis_index("s") * _NC + lax.axis_index("c")
        base = wid * rpw
        rh = rpw // 2
        halves = []
        for h, sem in ((0, sem0), (1, sem1)):
            halves.append([
                pltpu.async_copy(
                    keys_hbm.at[e, pl.ds(base + h * rh, rh)],
                    kbuf.at[pl.ds(e * rpw + h * rh, rh)], sem)
                for e in range(nexp)
            ])
        def load_sorted8(b, g0, gi):
            s = [kbuf[pl.ds((b * _TOP_K + j) * rpw + (g0 + gi) * _L, _L)]
                 for j in range(_TOP_K)]
            for (i, j) in _SORT8:
                hi = jnp.maximum(s[i], s[j])
                lo = jnp.minimum(s[i], s[j])
                s[i], s[j] = hi, lo
            return s

        def group_body(gb, carry):
            g0 = gb * _GI

            def ebatch(b, ts):
                new = []
                for gi, t in enumerate(ts):
                    s = load_sorted8(b, g0, gi)
                    m = [jnp.maximum(t[i], s[7 - i]) for i in range(_TOP_K)]
                    for (i, j) in _MERGE8:
                        hi = jnp.maximum(m[i], m[j])
                        lo = jnp.minimum(m[i], m[j])
                        m[i], m[j] = hi, lo
                    new.append(tuple(m))
                return tuple(new)

            init = tuple(tuple(load_sorted8(0, g0, gi)) for gi in range(_GI))
            ts = lax.fori_loop(1, nexp // _TOP_K, ebatch, init)
            for gi, t in enumerate(ts):
                opos = (g0 + gi) * _L
                for j in range(_TOP_K):
                    w = lax.bitcast_convert_type(t[j] & ~63, jnp.float32)
                    i = (nexp - 1) - (t[j] & 63)
                    wbuf[pl.ds(j * rpw + opos, _L)] = w
                    ibuf[pl.ds(j * rpw + opos, _L)] = i
            return carry

        half_blocks = groups // 2 // _GI
        for c in halves[0]:
            c.wait()
        lax.fori_loop(0, half_blocks, group_body, 0)
        for c in halves[1]:
            c.wait()
        lax.fori_loop(half_blocks, 2 * half_blocks, group_body, 0)
        for j in range(_TOP_K):
            pltpu.sync_copy(wbuf.at[pl.ds(j * rpw, rpw)],
                            w_hbm.at[j, pl.ds(base, rpw)])
            pltpu.sync_copy(ibuf.at[pl.ds(j * rpw, rpw)],
                            i_hbm.at[j, pl.ds(base, rpw)])

    return k(keys_t)


def kernel(x, weight):
    wt = weight.T
    ws, idxs = [], []
    for c in range(_CHUNKS):
        keys_t = _tc_keys_t(x, wt, c, _CHUNKS)
        w_t, i_t = _sc_topk_t(keys_t)
        ws.append(w_t.T)
        idxs.append(i_t.T)
    return jnp.concatenate(ws, axis=0), jnp.concatenate(idxs, axis=0)


# speedup vs baseline: 1.0801x; 1.0011x over previous
import functools

import jax
import jax.numpy as jnp
from jax import lax
from jax.experimental import pallas as pl
from jax.experimental.pallas import tpu as pltpu
from jax.experimental.pallas import tpu_sc as plsc

_TOP_K = 8
_BT = 2048
_CHUNKS = 1
_NC, _NS, _L = 2, 16, 16
_NW = _NC * _NS
_GI = 2

_SORT8 = [(0, 1), (2, 3), (4, 5), (6, 7), (0, 2), (1, 3), (4, 6), (5, 7),
          (1, 2), (5, 6), (0, 4), (1, 5), (2, 6), (3, 7), (2, 4), (3, 5),
          (1, 2), (3, 4), (5, 6)]
_MERGE8 = [(0, 4), (1, 5), (2, 6), (3, 7), (0, 2), (1, 3), (4, 6), (5, 7),
           (0, 1), (2, 3), (4, 5), (6, 7)]


def _keys_block(x_ref, wt_ref, k_out_ref):
    logits = lax.dot_general(
        x_ref[...], wt_ref[...], (((1,), (0,)), ((), ())),
        preferred_element_type=jnp.float32,
    )
    m = jnp.max(logits, axis=1, keepdims=True)
    e = jnp.exp(logits - m)
    s = jnp.sum(e, axis=1, keepdims=True)
    p = e / s
    ncol = logits.shape[1]
    col = lax.broadcasted_iota(jnp.int32, logits.shape, 1)
    keys = (lax.bitcast_convert_type(p, jnp.int32) & ~63) | (ncol - 1 - col)
    k_out_ref[...] = keys.T


def _tc_keys_t(x, wt, chunk, nchunks):
    tokens, hidden = x.shape
    nexp = wt.shape[1]
    per_chunk = tokens // nchunks
    blocks = per_chunk // _BT
    base = chunk * blocks
    return pl.pallas_call(
        _keys_block,
        grid=(blocks,),
        in_specs=[
            pl.BlockSpec((_BT, hidden), lambda i: (base + i, 0)),
            pl.BlockSpec((hidden, nexp), lambda i: (0, 0)),
        ],
        out_specs=pl.BlockSpec((nexp, _BT), lambda i: (0, i)),
        out_shape=jax.ShapeDtypeStruct((nexp, per_chunk), jnp.int32),
        compiler_params=pltpu.CompilerParams(
            dimension_semantics=("arbitrary",),
        ),
    )(x, wt)


def _sc_topk_t(keys_t):
    nexp, rows = keys_t.shape
    rpw = rows // _NW
    groups = rpw // _L
    mesh = plsc.VectorSubcoreMesh(core_axis_name="c", subcore_axis_name="s")

    @functools.partial(
        pl.kernel,
        mesh=mesh,
        out_type=[
            jax.ShapeDtypeStruct((_TOP_K, rows), jnp.float32),
            jax.ShapeDtypeStruct((_TOP_K, rows), jnp.int32),
        ],
        scratch_types=[
            pltpu.VMEM((nexp * rpw,), jnp.int32),
            pltpu.VMEM((_TOP_K * rpw,), jnp.float32),
            pltpu.VMEM((_TOP_K * rpw,), jnp.int32),
            pltpu.SemaphoreType.DMA,
            pltpu.SemaphoreType.DMA,
        ],
    )
    def k(keys_hbm, w_hbm, i_hbm, kbuf, wbuf, ibuf, sem0, sem1):
        wid = lax.axis_index("s") * _NC + lax.axis_index("c")
        base = wid * rpw
        rh = rpw // 2
        halves = []
        for h, sem in ((0, sem0), (1, sem1)):
            halves.append([
                pltpu.async_copy(
                    keys_hbm.at[e, pl.ds(base + h * rh, rh)],
                    kbuf.at[pl.ds(e * rpw + h * rh, rh)], sem)
                for e in range(nexp)
            ])
        def load_sorted8(b, g0, gi):
            s = [kbuf[pl.ds((b * _TOP_K + j) * rpw + (g0 + gi) * _L, _L)]
                 for j in range(_TOP_K)]
            for (i, j) in _SORT8:
                hi = jnp.maximum(s[i], s[j])
                lo = jnp.minimum(s[i], s[j])
                s[i], s[j] = hi, lo
            return s

        def group_body(gb, carry):
            g0 = gb * _GI

            def ebatch(b, ts):
                new = []
                for gi, t in enumerate(ts):
                    s = load_sorted8(b, g0, gi)
                    m = [jnp.maximum(t[i], s[7 - i]) for i in range(_TOP_K)]
                    for (i, j) in _MERGE8:
                        hi = jnp.maximum(m[i], m[j])
                        lo = jnp.minimum(m[i], m[j])
                        m[i], m[j] = hi, lo
                    new.append(tuple(m))
                return tuple(new)

            init = tuple(tuple(load_sorted8(0, g0, gi)) for gi in range(_GI))
            ts = lax.fori_loop(1, nexp // _TOP_K, ebatch, init)
            for gi, t in enumerate(ts):
                opos = (g0 + gi) * _L
                for j in range(_TOP_K):
                    w = lax.bitcast_convert_type(t[j] & ~63, jnp.float32)
                    i = (nexp - 1) - (t[j] & 63)
                    wbuf[pl.ds(j * rpw + opos, _L)] = w
                    ibuf[pl.ds(j * rpw + opos, _L)] = i
            return carry

        half_blocks = groups // 2 // _GI
        for c in halves[0]:
            c.wait()
        lax.fori_loop(0, half_blocks, group_body, 0)
        for c in halves[1]:
            c.wait()
        lax.fori_loop(half_blocks, 2 * half_blocks, group_body, 0)
        for j in range(_TOP_K):
            pltpu.sync_copy(wbuf.at[pl.ds(j * rpw, rpw)],
                            w_hbm.at[j, pl.ds(base, rpw)])
            pltpu.sync_copy(ibuf.at[pl.ds(j * rpw, rpw)],
                            i_hbm.at[j, pl.ds(base, rpw)])

    return k(keys_t)


def kernel(x, weight):
    wt = weight.T
    keys_t = _tc_keys_t(x, wt, 0, _CHUNKS)
    w_t, i_t = _sc_topk_t(keys_t)
    return w_t.T, i_t.T
